# baseline (device time: 19293 ns/iter reference)
import os

import jax
import jax.numpy as jnp
from jax import lax
from jax.experimental import pallas as pl
from jax.experimental.pallas import tpu as pltpu

N_DEV = 32
_VARIANT = os.environ.get("KV", "full")


def kernel(x, w_mat):
    m_per, k_dim = x.shape
    _, n = w_mat.shape
    n_per = n // N_DEV

    def body(x_ref, w_ref, out_ref, y_ref, send_sems, recv_sems):
        my = lax.axis_index("i")

        if _VARIANT != "nocomm":
            barrier_sem = pltpu.get_barrier_semaphore()
            n_bar = 2 if _VARIANT == "weakbar" else N_DEV - 1
            peers = [1, N_DEV - 1] if _VARIANT == "weakbar" else range(1, N_DEV)
            for p in peers:
                pl.semaphore_signal(
                    barrier_sem,
                    inc=1,
                    device_id=(lax.rem(my + p, N_DEV),),
                    device_id_type=pl.DeviceIdType.MESH,
                )

        if _VARIANT == "nogemm":
            for d in range(N_DEV):
                y_ref[d] = jnp.zeros((m_per, n_per), jnp.float32)
        else:
            y = jnp.dot(
                x_ref[...], w_ref[...], preferred_element_type=jnp.float32
            )
            for d in range(N_DEV):
                y_ref[d] = y[:, d * n_per:(d + 1) * n_per]

        out_ref[pl.ds(my * m_per, m_per), :] = y_ref[my]

        if _VARIANT == "nocomm":
            return

        pl.semaphore_wait(barrier_sem, n_bar)

        if _VARIANT in ("flows7", "flows3"):
            nf = 7 if _VARIANT == "flows7" else 3
            blk = 4 if _VARIANT == "flows7" else 10
            rdmas = []
            for kk in range(1, nf + 1):
                d = lax.rem(my + kk, N_DEV)
                rdma = pltpu.make_async_remote_copy(
                    src_ref=y_ref.at[pl.ds((kk - 1) * blk, blk)],
                    dst_ref=y_ref.at[pl.ds((kk - 1) * blk, blk)],
                    send_sem=send_sems.at[kk - 1],
                    recv_sem=recv_sems.at[kk - 1],
                    device_id=(d,),
                    device_id_type=pl.DeviceIdType.MESH,
                )
                rdma.start()
                rdmas.append(rdma)
            for rdma in rdmas:
                rdma.wait()
            return

        rdmas = []
        for kk in range(1, N_DEV):
            d = lax.rem(my + kk, N_DEV)
            rdma = pltpu.make_async_remote_copy(
                src_ref=y_ref.at[d],
                dst_ref=out_ref.at[pl.ds(my * m_per, m_per), :],
                send_sem=send_sems.at[kk - 1],
                recv_sem=recv_sems.at[kk - 1],
                device_id=(d,),
                device_id_type=pl.DeviceIdType.MESH,
            )
            rdma.start()
            rdmas.append(rdma)
        for rdma in rdmas:
            rdma.wait()

    return pl.pallas_call(
        body,
        out_shape=jax.ShapeDtypeStruct((N_DEV * m_per, n_per), jnp.float32),
        in_specs=[
            pl.BlockSpec(memory_space=pltpu.VMEM),
            pl.BlockSpec(memory_space=pltpu.VMEM),
        ],
        out_specs=pl.BlockSpec(memory_space=pltpu.VMEM),
        scratch_shapes=[
            pltpu.VMEM((N_DEV, m_per, n_per), jnp.float32),
            pltpu.SemaphoreType.DMA((N_DEV - 1,)),
            pltpu.SemaphoreType.DMA((N_DEV - 1,)),
        ],
        compiler_params=(
            None
            if _VARIANT == "nocomm"
            else pltpu.CompilerParams(collective_id=0)
        ),
    )(x, w_mat)


# device time: 14381 ns/iter; 1.3416x vs baseline; 1.3416x over previous
import os

import jax
import jax.numpy as jnp
from jax import lax
from jax.experimental import pallas as pl
from jax.experimental.pallas import tpu as pltpu

N_DEV = 32
_VARIANT = os.environ.get("KV", "full")


def kernel(x, w_mat):
    m_per, k_dim = x.shape
    _, n = w_mat.shape
    n_per = n // N_DEV

    def body(x_ref, w_ref, out_ref, y_ref, send_sems, recv_sems):
        my = lax.axis_index("i")

        if _VARIANT != "nocomm":
            barrier_sem = pltpu.get_barrier_semaphore()
            weak = _VARIANT in ("weakbar", "minflow")
            n_bar = 2 if weak else N_DEV - 1
            peers = [1, N_DEV - 1] if weak else range(1, N_DEV)
            for p in peers:
                pl.semaphore_signal(
                    barrier_sem,
                    inc=1,
                    device_id=(lax.rem(my + p, N_DEV),),
                    device_id_type=pl.DeviceIdType.MESH,
                )

        if _VARIANT == "minflow":
            pl.semaphore_wait(barrier_sem, n_bar)
            rdma = pltpu.make_async_remote_copy(
                src_ref=y_ref.at[0],
                dst_ref=y_ref.at[1],
                send_sem=send_sems.at[0],
                recv_sem=recv_sems.at[0],
                device_id=(lax.rem(my + 1, N_DEV),),
                device_id_type=pl.DeviceIdType.MESH,
            )
            rdma.start()
            rdma.wait()
            out_ref[pl.ds(my * m_per, m_per), :] = y_ref[1]
            return

        if _VARIANT == "nogemm":
            for d in range(N_DEV):
                y_ref[d] = jnp.zeros((m_per, n_per), jnp.float32)
        else:
            y = jnp.dot(
                x_ref[...], w_ref[...], preferred_element_type=jnp.float32
            )
            for d in range(N_DEV):
                y_ref[d] = y[:, d * n_per:(d + 1) * n_per]

        out_ref[pl.ds(my * m_per, m_per), :] = y_ref[my]

        if _VARIANT == "nocomm":
            return

        pl.semaphore_wait(barrier_sem, n_bar)

        if _VARIANT in ("flows7", "flows3"):
            nf = 7 if _VARIANT == "flows7" else 3
            blk = 4 if _VARIANT == "flows7" else 10
            rdmas = []
            for kk in range(1, nf + 1):
                d = lax.rem(my + kk, N_DEV)
                rdma = pltpu.make_async_remote_copy(
                    src_ref=y_ref.at[pl.ds((kk - 1) * blk, blk)],
                    dst_ref=y_ref.at[pl.ds((kk - 1) * blk, blk)],
                    send_sem=send_sems.at[kk - 1],
                    recv_sem=recv_sems.at[kk - 1],
                    device_id=(d,),
                    device_id_type=pl.DeviceIdType.MESH,
                )
                rdma.start()
                rdmas.append(rdma)
            for rdma in rdmas:
                rdma.wait()
            return

        rdmas = []
        for kk in range(1, N_DEV):
            d = lax.rem(my + kk, N_DEV)
            rdma = pltpu.make_async_remote_copy(
                src_ref=y_ref.at[d],
                dst_ref=out_ref.at[pl.ds(my * m_per, m_per), :],
                send_sem=send_sems.at[kk - 1],
                recv_sem=recv_sems.at[kk - 1],
                device_id=(d,),
                device_id_type=pl.DeviceIdType.MESH,
            )
            rdma.start()
            rdmas.append(rdma)
        for rdma in rdmas:
            rdma.wait()

    return pl.pallas_call(
        body,
        out_shape=jax.ShapeDtypeStruct((N_DEV * m_per, n_per), jnp.float32),
        in_specs=[
            pl.BlockSpec(memory_space=pltpu.VMEM),
            pl.BlockSpec(memory_space=pltpu.VMEM),
        ],
        out_specs=pl.BlockSpec(memory_space=pltpu.VMEM),
        scratch_shapes=[
            pltpu.VMEM((N_DEV, m_per, n_per), jnp.float32),
            pltpu.SemaphoreType.DMA((N_DEV - 1,)),
            pltpu.SemaphoreType.DMA((N_DEV - 1,)),
        ],
        compiler_params=(
            None
            if _VARIANT == "nocomm"
            else pltpu.CompilerParams(collective_id=0)
        ),
    )(x, w_mat)
